# Initial kernel scaffold; baseline (speedup 1.0000x reference)
import jax
import jax.numpy as jnp
from jax import lax
from jax.experimental import pallas as pl
from jax.experimental.pallas import tpu as pltpu
from jax.experimental.pallas import tpu_sc as plsc

B = 16384
L = 200
D = 32
NC = 2
NS = 16
NW = NC * NS
PB = B // NW
G = 4
NG = PB // G
RG = G * L


def _fism_body(his_flat_hbm, pre_hbm, users_hbm, scale_hbm,
               his_tab_hbm, item_tab_hbm, ubias_hbm, ibias_hbm,
               out_hbm,
               idx_v, rows_v, pre_v, users_v, item_rows, scale_v,
               ubias_v, ibias_v, out_v, sem, sem2):
    wid = lax.axis_index("s") * NC + lax.axis_index("c")
    base = wid * PB

    pltpu.sync_copy(pre_hbm.at[pl.ds(base, PB)], pre_v)
    pltpu.sync_copy(users_hbm.at[pl.ds(base, PB)], users_v)
    pltpu.sync_copy(scale_hbm.at[pl.ds(base, PB)], scale_v)

    cp_items = pltpu.async_copy(item_tab_hbm.at[pre_v], item_rows, sem2)
    cp_ub = pltpu.async_copy(ubias_hbm.at[users_v], ubias_v, sem2)
    cp_ib = pltpu.async_copy(ibias_hbm.at[pre_v], ibias_v, sem2)
    cp_items.wait()
    cp_ub.wait()
    cp_ib.wait()

    @pl.loop(0, NG)
    def _(g):
        r0 = (base + g * G) * L
        pltpu.sync_copy(his_flat_hbm.at[pl.ds(r0, RG)], idx_v)
        pltpu.async_copy(his_tab_hbm.at[idx_v], rows_v, sem).wait()
        for u in range(G):
            uu = g * G + u
            acc = [jnp.zeros((16,), jnp.float32) for _ in range(8)]
            for j in range(L):
                c = j % 4
                acc[c] = acc[c] + rows_v[u * L + j, pl.ds(0, 16)]
                acc[4 + c] = acc[4 + c] + rows_v[u * L + j, pl.ds(16, 16)]
            lo = (acc[0] + acc[1]) + (acc[2] + acc[3])
            hi = (acc[4] + acc[5]) + (acc[6] + acc[7])
            prod = (lo * item_rows[uu, pl.ds(0, 16)]
                    + hi * item_rows[uu, pl.ds(16, 16)])
            s = jnp.sum(prod)
            out_v[uu] = s * scale_v[uu] + ubias_v[uu] + ibias_v[uu]

    pltpu.sync_copy(out_v, out_hbm.at[pl.ds(base, PB)])


def kernel(users, his_items, his_lens, pre_items, his_emb_table,
           item_emb_table, user_bias_table, item_bias_table):
    scale = jnp.power(his_lens, -0.5).astype(jnp.float32)
    his_flat = his_items.reshape(B * L).astype(jnp.int32)
    pre = pre_items.astype(jnp.int32)
    usr = users.astype(jnp.int32)
    ub = user_bias_table.reshape(-1)
    ib = item_bias_table.reshape(-1)
    mesh = plsc.VectorSubcoreMesh(core_axis_name="c", subcore_axis_name="s")
    fn = pl.kernel(
        _fism_body,
        out_type=jax.ShapeDtypeStruct((B,), jnp.float32),
        mesh=mesh,
        scratch_types=[
            pltpu.VMEM((RG,), jnp.int32),
            pltpu.VMEM((RG, D), jnp.float32),
            pltpu.VMEM((PB,), jnp.int32),
            pltpu.VMEM((PB,), jnp.int32),
            pltpu.VMEM((PB, D), jnp.float32),
            pltpu.VMEM((PB,), jnp.float32),
            pltpu.VMEM((PB,), jnp.float32),
            pltpu.VMEM((PB,), jnp.float32),
            pltpu.VMEM((PB,), jnp.float32),
            pltpu.SemaphoreType.DMA,
            pltpu.SemaphoreType.DMA,
        ],
    )
    return fn(his_flat, pre, usr, scale, his_emb_table, item_emb_table, ub, ib)


# speedup vs baseline: 6.8002x; 6.8002x over previous
import jax
import jax.numpy as jnp
from jax import lax
from jax.experimental import pallas as pl
from jax.experimental.pallas import tpu as pltpu
from jax.experimental.pallas import tpu_sc as plsc

B = 16384
L = 200
D = 32
NC = 2
NS = 16
NW = NC * NS
PB = B // NW
G = 4
RG = G * L
UB = 16
NB = PB // UB


def _fism_body(his_flat_hbm, pre_hbm, users_hbm, scale_hbm,
               his_tab_hbm, item_tab_hbm, ubias_hbm, ibias_hbm,
               out_hbm,
               idx_v, rows_v, pre_v, users_v, item_rows, scale_v,
               ubias_v, ibias_v, out_v, prod_buf, sem, sem2):
    wid = lax.axis_index("s") * NC + lax.axis_index("c")
    base = wid * PB

    pltpu.sync_copy(pre_hbm.at[pl.ds(base, PB)], pre_v)
    pltpu.sync_copy(users_hbm.at[pl.ds(base, PB)], users_v)
    pltpu.sync_copy(scale_hbm.at[pl.ds(base, PB)], scale_v)

    cp_items = pltpu.async_copy(item_tab_hbm.at[pre_v], item_rows, sem2)
    cp_ub = pltpu.async_copy(ubias_hbm.at[users_v], ubias_v, sem2)
    cp_ib = pltpu.async_copy(ibias_hbm.at[pre_v], ibias_v, sem2)
    cp_items.wait()
    cp_ub.wait()
    cp_ib.wait()

    lane = lax.iota(jnp.int32, 16)

    @pl.loop(0, NB)
    def _(blk):
        b0 = blk * UB

        @pl.loop(0, UB // G)
        def _(s):
            u0 = b0 + s * G
            r0 = (base + u0) * L
            pltpu.sync_copy(his_flat_hbm.at[pl.ds(r0, RG)], idx_v)
            pltpu.async_copy(his_tab_hbm.at[idx_v], rows_v, sem).wait()
            for u in range(G):
                uu = u0 + u
                acc = [jnp.zeros((16,), jnp.float32) for _ in range(8)]
                for j in range(L):
                    c = j % 4
                    acc[c] = acc[c] + rows_v[u * L + j, pl.ds(0, 16)]
                    acc[4 + c] = acc[4 + c] + rows_v[u * L + j, pl.ds(16, 16)]
                lo = (acc[0] + acc[1]) + (acc[2] + acc[3])
                hi = (acc[4] + acc[5]) + (acc[6] + acc[7])
                prod = (lo * item_rows[uu, pl.ds(0, 16)]
                        + hi * item_rows[uu, pl.ds(16, 16)])
                prod_buf[pl.ds((s * G + u) * 16, 16)] = prod

        rating = jnp.zeros((16,), jnp.float32)
        lane16 = lane * 16
        for d in range(16):
            col = plsc.load_gather(prod_buf, [lane16 + d])
            rating = rating + col
        rating = (rating * scale_v[pl.ds(b0, UB)]
                  + ubias_v[pl.ds(b0, UB)] + ibias_v[pl.ds(b0, UB)])
        out_v[pl.ds(b0, UB)] = rating

    pltpu.sync_copy(out_v, out_hbm.at[pl.ds(base, PB)])


def kernel(users, his_items, his_lens, pre_items, his_emb_table,
           item_emb_table, user_bias_table, item_bias_table):
    scale = jnp.power(his_lens, -0.5).astype(jnp.float32)
    his_flat = his_items.reshape(B * L).astype(jnp.int32)
    pre = pre_items.astype(jnp.int32)
    usr = users.astype(jnp.int32)
    ub = user_bias_table.reshape(-1)
    ib = item_bias_table.reshape(-1)
    mesh = plsc.VectorSubcoreMesh(core_axis_name="c", subcore_axis_name="s")
    fn = pl.kernel(
        _fism_body,
        out_type=jax.ShapeDtypeStruct((B,), jnp.float32),
        mesh=mesh,
        compiler_params=pltpu.CompilerParams(
            needs_layout_passes=False, use_tc_tiling_on_sc=False),
        scratch_types=[
            pltpu.VMEM((RG,), jnp.int32),
            pltpu.VMEM((RG, D), jnp.float32),
            pltpu.VMEM((PB,), jnp.int32),
            pltpu.VMEM((PB,), jnp.int32),
            pltpu.VMEM((PB, D), jnp.float32),
            pltpu.VMEM((PB,), jnp.float32),
            pltpu.VMEM((PB,), jnp.float32),
            pltpu.VMEM((PB,), jnp.float32),
            pltpu.VMEM((PB,), jnp.float32),
            pltpu.VMEM((UB * 16,), jnp.float32),
            pltpu.SemaphoreType.DMA,
            pltpu.SemaphoreType.DMA,
        ],
    )
    return fn(his_flat, pre, usr, scale, his_emb_table, item_emb_table, ub, ib)


# speedup vs baseline: 10.7968x; 1.5877x over previous
import jax
import jax.numpy as jnp
from jax import lax
from jax.experimental import pallas as pl
from jax.experimental.pallas import tpu as pltpu
from jax.experimental.pallas import tpu_sc as plsc

B = 16384
L = 200
D = 32
NC = 2
NS = 16
NW = NC * NS
PB = B // NW
G = 4
RG = G * L
UB = 16
HU = PB // 2
HB = HU // UB


def _fism_body(his_flat_hbm, pre_hbm, users_hbm, scale_hbm,
               his_tab_hbm, item_tab_hbm, ubias_hbm, ibias_hbm,
               out_hbm,
               idx_half, rows0, rows1, pre_v, users_v, item_rows, scale_v,
               ubias_v, ibias_v, out_v, prod_buf,
               semr0, semr1, sem2):
    wid = lax.axis_index("s") * NC + lax.axis_index("c")
    base = wid * PB
    rows = (rows0, rows1)
    sems = (semr0, semr1)

    pltpu.sync_copy(pre_hbm.at[pl.ds(base, PB)], pre_v)
    pltpu.sync_copy(users_hbm.at[pl.ds(base, PB)], users_v)
    pltpu.sync_copy(scale_hbm.at[pl.ds(base, PB)], scale_v)
    cp_items = pltpu.async_copy(item_tab_hbm.at[pre_v], item_rows, sem2)
    cp_ub = pltpu.async_copy(ubias_hbm.at[users_v], ubias_v, sem2)
    cp_ib = pltpu.async_copy(ibias_hbm.at[pre_v], ibias_v, sem2)

    lane16 = lax.iota(jnp.int32, 16) * 16

    def fire(sg, b):
        pltpu.async_copy(
            his_tab_hbm.at[idx_half.at[pl.ds(sg * RG, RG)]], rows[b], sems[b])

    def wait(b):
        pltpu.make_async_copy(
            his_tab_hbm.at[idx_half.at[pl.ds(0, RG)]], rows[b], sems[b]).wait()

    cp_items.wait()
    cp_ub.wait()
    cp_ib.wait()

    @pl.loop(0, 2)
    def _(h):
        ho = h * HU
        pltpu.sync_copy(his_flat_hbm.at[pl.ds((base + ho) * L, HU * L)],
                        idx_half)
        fire(0, 0)

        @pl.loop(0, HB)
        def _(blk):
            for s in range(4):
                b = s % 2
                sg = blk * 4 + s
                if s < 3:
                    fire(sg + 1, 1 - b)
                else:
                    @pl.when(blk < HB - 1)
                    def _():
                        fire(sg + 1, 1 - b)
                wait(b)
                rv = rows[b]

                @pl.loop(0, G)
                def _(u):
                    init = (jnp.zeros((16,), jnp.float32),) * 8

                    @pl.loop(0, L, step=8, unroll=5, init_carry=init)
                    def pool(j, accs):
                        accs = list(accs)
                        for k in range(8):
                            r = u * L + j + k
                            c = k % 4
                            accs[c] = accs[c] + rv[r, pl.ds(0, 16)]
                            accs[4 + c] = accs[4 + c] + rv[r, pl.ds(16, 16)]
                        return tuple(accs)

                    lo = (pool[0] + pool[1]) + (pool[2] + pool[3])
                    hi = (pool[4] + pool[5]) + (pool[6] + pool[7])
                    uu = ho + blk * UB + s * G + u
                    prod = (lo * item_rows[uu, pl.ds(0, 16)]
                            + hi * item_rows[uu, pl.ds(16, 16)])
                    prod_buf[pl.ds((s * G + u) * 16, 16)] = prod

            rating = jnp.zeros((16,), jnp.float32)
            for d in range(16):
                rating = rating + plsc.load_gather(prod_buf, [lane16 + d])
            b0 = ho + blk * UB
            rating = (rating * scale_v[pl.ds(b0, UB)]
                      + ubias_v[pl.ds(b0, UB)] + ibias_v[pl.ds(b0, UB)])
            out_v[pl.ds(b0, UB)] = rating

    pltpu.sync_copy(out_v, out_hbm.at[pl.ds(base, PB)])


def kernel(users, his_items, his_lens, pre_items, his_emb_table,
           item_emb_table, user_bias_table, item_bias_table):
    scale = jnp.power(his_lens, -0.5).astype(jnp.float32)
    his_flat = his_items.reshape(B * L).astype(jnp.int32)
    pre = pre_items.astype(jnp.int32)
    usr = users.astype(jnp.int32)
    ub = user_bias_table.reshape(-1)
    ib = item_bias_table.reshape(-1)
    mesh = plsc.VectorSubcoreMesh(core_axis_name="c", subcore_axis_name="s")
    fn = pl.kernel(
        _fism_body,
        out_type=jax.ShapeDtypeStruct((B,), jnp.float32),
        mesh=mesh,
        compiler_params=pltpu.CompilerParams(
            needs_layout_passes=False, use_tc_tiling_on_sc=False),
        scratch_types=[
            pltpu.VMEM((HU * L,), jnp.int32),
            pltpu.VMEM((RG, D), jnp.float32),
            pltpu.VMEM((RG, D), jnp.float32),
            pltpu.VMEM((PB,), jnp.int32),
            pltpu.VMEM((PB,), jnp.int32),
            pltpu.VMEM((PB, D), jnp.float32),
            pltpu.VMEM((PB,), jnp.float32),
            pltpu.VMEM((PB,), jnp.float32),
            pltpu.VMEM((PB,), jnp.float32),
            pltpu.VMEM((PB,), jnp.float32),
            pltpu.VMEM((UB * 16,), jnp.float32),
            pltpu.SemaphoreType.DMA,
            pltpu.SemaphoreType.DMA,
            pltpu.SemaphoreType.DMA,
        ],
    )
    return fn(his_flat, pre, usr, scale, his_emb_table, item_emb_table, ub, ib)
